# baseline (speedup 1.0000x reference)
import functools

import jax
import jax.numpy as jnp
from jax import lax
from jax.experimental import pallas as pl
from jax.experimental.pallas import tpu as pltpu
from jax.experimental.pallas import tpu_sc as plsc

N_NODES = 10000
D = 128
DH = D // 2

NUM_CORES = 2
NUM_SUBCORES = 16
K = 80
CH = 25

PAD_ROWS = 10112
ROWS_PER_SUB = PAD_ROWS // NUM_SUBCORES


def _sc_spmm(NB):
    mesh = plsc.VectorSubcoreMesh(
        core_axis_name="c", subcore_axis_name="s",
        num_cores=NUM_CORES, num_subcores=NUM_SUBCORES)

    @functools.partial(
        pl.kernel,
        out_type=jax.ShapeDtypeStruct((NUM_CORES, PAD_ROWS, DH), jnp.float32),
        mesh=mesh,
        scratch_types=dict(
            sbuf=pltpu.VMEM((4, K), jnp.int32),
            dst_c=pltpu.VMEM((2, CH, K), jnp.int32),
            rows_v=pltpu.VMEM((4, K, DH), jnp.float32),
            xsh=pltpu.VMEM_SHARED((PAD_ROWS, DH), jnp.float32),
            acc=pltpu.VMEM_SHARED((PAD_ROWS, DH), jnp.float32),
            sem_i=pltpu.SemaphoreType.DMA((4,)),
            sem_r=pltpu.SemaphoreType.DMA((4,)),
            sem_w=pltpu.SemaphoreType.DMA((4,)),
            sem_d=pltpu.SemaphoreType.DMA((2,)),
        ),
        compiler_params=pltpu.CompilerParams(use_tc_tiling_on_sc=False),
    )
    def spmm(x_hbm, src_hbm, dst_hbm, out_hbm,
             sbuf, dst_c, rows_v, xsh, acc, sem_i, sem_r, sem_w, sem_d):
        NCH = NB // CH
        c = lax.axis_index("c")
        s = lax.axis_index("s")

        col = pl.ds(c * DH, DH)
        last_rows = N_NODES - (NUM_SUBCORES - 1) * ROWS_PER_SUB

        @pl.when(s < NUM_SUBCORES - 1)
        def _():
            rows = pl.ds(s * ROWS_PER_SUB, ROWS_PER_SUB)
            pltpu.sync_copy(x_hbm.at[rows, col], xsh.at[rows])
            pltpu.sync_copy(x_hbm.at[rows, col], acc.at[rows])

        @pl.when(s == NUM_SUBCORES - 1)
        def _():
            rows = pl.ds((NUM_SUBCORES - 1) * ROWS_PER_SUB, last_rows)
            pltpu.sync_copy(x_hbm.at[rows, col], xsh.at[rows])
            pltpu.sync_copy(x_hbm.at[rows, col], acc.at[rows])

        pltpu.sync_copy(dst_hbm.at[s].at[pl.ds(0, CH)], dst_c.at[0])
        plsc.subcore_barrier()

        def idx_copy(j, slot):
            return pltpu.make_async_copy(src_hbm.at[s].at[j], sbuf.at[slot],
                                         sem_i.at[slot])

        def gather(j, slot):
            return pltpu.make_async_copy(xsh.at[sbuf.at[slot]],
                                         rows_v.at[slot], sem_r.at[slot])

        def scatter(qb, jj, slot):
            return pltpu.make_async_copy(
                rows_v.at[slot], acc.at[dst_c.at[qb].at[jj]], sem_w.at[slot])

        pltpu.sync_copy(src_hbm.at[s].at[0], sbuf.at[0])
        idx_copy(1, 1).start()
        idx_copy(2, 2).start()
        gather(0, 0).start()
        idx_copy(1, 1).wait()
        gather(1, 1).start()

        def chunk(q, _):
            qb = lax.rem(q, 2)

            @pl.when(q + 1 < NCH)
            def _():
                pltpu.async_copy(dst_hbm.at[s].at[pl.ds((q + 1) * CH, CH)],
                                 dst_c.at[lax.rem(q + 1, 2)],
                                 sem_d.at[lax.rem(q + 1, 2)])

            @pl.when(q > 0)
            def _():
                pltpu.make_async_copy(dst_hbm.at[s].at[pl.ds(q * CH, CH)],
                                      dst_c.at[qb], sem_d.at[qb]).wait()

            def step(jj, _):
                g = q * CH + jj
                b = lax.rem(g, 4)

                gather(g, b).wait()
                scatter(qb, jj, b).start(add=True)

                @pl.when(jj >= 2)
                def _():
                    scatter(qb, jj - 2, lax.rem(g + 2, 4)).wait()

                @pl.when(g + 2 < NB)
                def _():
                    idx_copy(g + 2, lax.rem(g + 2, 4)).wait()
                    gather(g + 2, lax.rem(g + 2, 4)).start()

                @pl.when(g + 3 < NB)
                def _():
                    idx_copy(g + 3, lax.rem(g + 3, 4)).start()

                return 0

            lax.fori_loop(0, CH, step, 0)
            scatter(qb, CH - 2, lax.rem(q * CH + CH - 2, 4)).wait()
            scatter(qb, CH - 1, lax.rem(q * CH + CH - 1, 4)).wait()
            return 0

        lax.fori_loop(0, NCH, chunk, 0)
        plsc.subcore_barrier()

        rs = pl.ds(s * ROWS_PER_SUB, ROWS_PER_SUB)
        pltpu.sync_copy(acc.at[rs], out_hbm.at[c].at[rs])

    return spmm


def _tc_body(agg_ref, w_ref, o_ref):
    xa = jnp.concatenate([agg_ref[0], agg_ref[1]], axis=1)
    o_ref[...] = jnp.dot(xa, w_ref[...], preferred_element_type=jnp.float32)


def kernel(X, edge_index, weight):
    E = edge_index.shape[1]
    NB = E // (NUM_SUBCORES * K)

    src3 = edge_index[0].reshape(NUM_SUBCORES, NB, K)
    dst3 = edge_index[1].reshape(NUM_SUBCORES, NB, K)

    agg = _sc_spmm(NB)(X, src3, dst3)

    n = X.shape[0]
    bm = 1000
    out = pl.pallas_call(
        _tc_body,
        grid=(n // bm,),
        in_specs=[
            pl.BlockSpec((NUM_CORES, bm, DH), lambda i: (0, i, 0)),
            pl.BlockSpec((D, D), lambda i: (0, 0)),
        ],
        out_specs=pl.BlockSpec((bm, D), lambda i: (i, 0)),
        out_shape=jax.ShapeDtypeStruct((n, D), jnp.float32),
    )(agg, weight)
    return out


# speedup vs baseline: 1.0610x; 1.0610x over previous
import functools

import jax
import jax.numpy as jnp
from jax import lax
from jax.experimental import pallas as pl
from jax.experimental.pallas import tpu as pltpu
from jax.experimental.pallas import tpu_sc as plsc

N_NODES = 10000
D = 128
DH = D // 2

NUM_CORES = 2
NUM_SUBCORES = 16
K = 80
CH = 25

PAD_ROWS = 10112
ROWS_PER_SUB = PAD_ROWS // NUM_SUBCORES


def _sc_spmm(NB):
    mesh = plsc.VectorSubcoreMesh(
        core_axis_name="c", subcore_axis_name="s",
        num_cores=NUM_CORES, num_subcores=NUM_SUBCORES)

    @functools.partial(
        pl.kernel,
        out_type=jax.ShapeDtypeStruct((NUM_CORES, PAD_ROWS, DH), jnp.float32),
        mesh=mesh,
        scratch_types=dict(
            sbuf=pltpu.VMEM((4, K), jnp.int32),
            dst_c=pltpu.VMEM((2, CH, K), jnp.int32),
            rows_v=pltpu.VMEM((4, K, DH), jnp.float32),
            xsh=pltpu.VMEM_SHARED((PAD_ROWS, DH), jnp.float32),
            acc=pltpu.VMEM_SHARED((PAD_ROWS, DH), jnp.float32),
            sem_i=pltpu.SemaphoreType.DMA((4,)),
            sem_r=pltpu.SemaphoreType.DMA((4,)),
            sem_w=pltpu.SemaphoreType.DMA((4,)),
            sem_d=pltpu.SemaphoreType.DMA((2,)),
        ),
        compiler_params=pltpu.CompilerParams(use_tc_tiling_on_sc=False),
    )
    def spmm(x_hbm, ei_hbm, out_hbm,
             sbuf, dst_c, rows_v, xsh, acc, sem_i, sem_r, sem_w, sem_d):
        src_hbm = ei_hbm.at[0]
        dst_hbm = ei_hbm.at[1]
        NCH = NB // CH
        c = lax.axis_index("c")
        s = lax.axis_index("s")

        col = pl.ds(c * DH, DH)
        last_rows = N_NODES - (NUM_SUBCORES - 1) * ROWS_PER_SUB

        @pl.when(s < NUM_SUBCORES - 1)
        def _():
            rows = pl.ds(s * ROWS_PER_SUB, ROWS_PER_SUB)
            pltpu.sync_copy(x_hbm.at[rows, col], xsh.at[rows])
            pltpu.sync_copy(x_hbm.at[rows, col], acc.at[rows])

        @pl.when(s == NUM_SUBCORES - 1)
        def _():
            rows = pl.ds((NUM_SUBCORES - 1) * ROWS_PER_SUB, last_rows)
            pltpu.sync_copy(x_hbm.at[rows, col], xsh.at[rows])
            pltpu.sync_copy(x_hbm.at[rows, col], acc.at[rows])

        pltpu.sync_copy(dst_hbm.at[s].at[pl.ds(0, CH)], dst_c.at[0])
        plsc.subcore_barrier()

        def idx_copy(j, slot):
            return pltpu.make_async_copy(src_hbm.at[s].at[j], sbuf.at[slot],
                                         sem_i.at[slot])

        def gather(j, slot):
            return pltpu.make_async_copy(xsh.at[sbuf.at[slot]],
                                         rows_v.at[slot], sem_r.at[slot])

        def scatter(qb, jj, slot):
            return pltpu.make_async_copy(
                rows_v.at[slot], acc.at[dst_c.at[qb].at[jj]], sem_w.at[slot])

        pltpu.sync_copy(src_hbm.at[s].at[0], sbuf.at[0])
        idx_copy(1, 1).start()
        idx_copy(2, 2).start()
        gather(0, 0).start()
        idx_copy(1, 1).wait()
        gather(1, 1).start()

        def chunk(q, _):
            qb = lax.rem(q, 2)

            @pl.when(q + 1 < NCH)
            def _():
                pltpu.async_copy(dst_hbm.at[s].at[pl.ds((q + 1) * CH, CH)],
                                 dst_c.at[lax.rem(q + 1, 2)],
                                 sem_d.at[lax.rem(q + 1, 2)])

            @pl.when(q > 0)
            def _():
                pltpu.make_async_copy(dst_hbm.at[s].at[pl.ds(q * CH, CH)],
                                      dst_c.at[qb], sem_d.at[qb]).wait()

            def step(jj, _):
                g = q * CH + jj
                b = lax.rem(g, 4)

                gather(g, b).wait()
                scatter(qb, jj, b).start(add=True)

                @pl.when(jj >= 2)
                def _():
                    scatter(qb, jj - 2, lax.rem(g + 2, 4)).wait()

                @pl.when(g + 2 < NB)
                def _():
                    idx_copy(g + 2, lax.rem(g + 2, 4)).wait()
                    gather(g + 2, lax.rem(g + 2, 4)).start()

                @pl.when(g + 3 < NB)
                def _():
                    idx_copy(g + 3, lax.rem(g + 3, 4)).start()

                return 0

            lax.fori_loop(0, CH, step, 0)
            scatter(qb, CH - 2, lax.rem(q * CH + CH - 2, 4)).wait()
            scatter(qb, CH - 1, lax.rem(q * CH + CH - 1, 4)).wait()
            return 0

        lax.fori_loop(0, NCH, chunk, 0)
        plsc.subcore_barrier()

        rs = pl.ds(s * ROWS_PER_SUB, ROWS_PER_SUB)
        pltpu.sync_copy(acc.at[rs], out_hbm.at[c].at[rs])

    return spmm


def _tc_body(agg_ref, w_ref, o_ref):
    xa = jnp.concatenate([agg_ref[0], agg_ref[1]], axis=1)
    o_ref[...] = jnp.dot(xa, w_ref[...], preferred_element_type=jnp.float32)


def kernel(X, edge_index, weight):
    E = edge_index.shape[1]
    NB = E // (NUM_SUBCORES * K)

    ei4 = edge_index.reshape(2, NUM_SUBCORES, NB, K)

    agg = _sc_spmm(NB)(X, ei4)

    n = X.shape[0]
    bm = 1000
    out = pl.pallas_call(
        _tc_body,
        grid=(n // bm,),
        in_specs=[
            pl.BlockSpec((NUM_CORES, bm, DH), lambda i: (0, i, 0)),
            pl.BlockSpec((D, D), lambda i: (0, 0)),
        ],
        out_specs=pl.BlockSpec((bm, D), lambda i: (i, 0)),
        out_shape=jax.ShapeDtypeStruct((n, D), jnp.float32),
    )(agg, weight)
    return out
